# Initial kernel scaffold; baseline (speedup 1.0000x reference)
import functools

import jax
import jax.numpy as jnp
from jax.experimental import pallas as pl
from jax.experimental.pallas import tpu as pltpu


def _relu(v):
    return jnp.maximum(v, 0.0)


def _body(x_ref, srcf_ref, dstf_ref, ef_ref, af_ref, bf_ref,
          WlT_ref, WrT_ref, bl_ref,
          W1a_ref, W1b_ref, W1c_ref, b11_ref, T12_ref, b12_ref,
          T13_ref, b13_ref, T14_ref, b14_ref,
          W2v_ref, W2u_ref, W2z_ref, W2e_ref, b21_ref, T22_ref, b22_ref,
          T23_ref, b23_ref, T24_ref, b24_ref,
          g0_ref, g1_ref, g2_ref, w00_ref, blin_ref,
          out_ref, A_s, Bz_s, P_s, *, E, N, EB):
    i = pl.program_id(0)

    @pl.when(i == 0)
    def _prologue():
        jN = jax.lax.broadcasted_iota(jnp.float32, (E, N), 1)
        oh_src = jnp.where(jN == srcf_ref[:], 1.0, 0.0)
        oh_dst = jnp.where(jN == dstf_ref[:], 1.0, 0.0)
        dn = (((0,), (0,)), ((), ()))
        M = jax.lax.dot_general(oh_dst, oh_src, dn,
                                preferred_element_type=jnp.float32)
        ones_e = jnp.ones((E, 1), dtype=jnp.float32)
        cnt = jax.lax.dot_general(oh_dst, ones_e, dn,
                                  preferred_element_type=jnp.float32)
        recip = 1.0 / jnp.maximum(cnt, 1.0)
        h = x_ref[:]
        for _ in range(3):
            mean = jnp.dot(M, h, preferred_element_type=jnp.float32) * recip
            h = _relu(jnp.dot(mean, WlT_ref[:], preferred_element_type=jnp.float32)
                      + bl_ref[:]
                      + jnp.dot(h, WrT_ref[:], preferred_element_type=jnp.float32))
        ve = jnp.dot(oh_src, h, preferred_element_type=jnp.float32)
        ue = jnp.dot(oh_dst, h, preferred_element_type=jnp.float32)
        efv = ef_ref[:]
        q = _relu(jnp.dot(ve, W1a_ref[:], preferred_element_type=jnp.float32)
                  + jnp.dot(ue, W1b_ref[:], preferred_element_type=jnp.float32)
                  + jnp.dot(efv, W1c_ref[:], preferred_element_type=jnp.float32)
                  + b11_ref[:])
        q = _relu(jnp.dot(q, T12_ref[:], preferred_element_type=jnp.float32) + b12_ref[:])
        q = _relu(jnp.dot(q, T13_ref[:], preferred_element_type=jnp.float32) + b13_ref[:])
        pno = jnp.dot(q, T14_ref[:], preferred_element_type=jnp.float32) + b14_ref[:]
        P_s[:] = pno * w00_ref[:] + blin_ref[:]
        A_s[:] = (jnp.dot(ve, W2v_ref[:], preferred_element_type=jnp.float32)
                  + jnp.dot(ue, W2u_ref[:], preferred_element_type=jnp.float32)
                  + jnp.dot(efv, W2e_ref[:], preferred_element_type=jnp.float32)
                  + b21_ref[:])
        Bz_s[:] = jnp.dot(h, W2z_ref[:], preferred_element_type=jnp.float32)

    Ab = A_s[pl.ds(i * EB, EB), :]
    X = _relu(Ab[:, None, :] + Bz_s[:][None, :, :])
    X2 = X.reshape(EB * N, 256)
    H2 = _relu(jnp.dot(X2, T22_ref[:], preferred_element_type=jnp.float32) + b22_ref[:])
    H3 = _relu(jnp.dot(H2, T23_ref[:], preferred_element_type=jnp.float32) + b23_ref[:])
    H3r = H3.reshape(EB, N, 64)
    jn = jax.lax.broadcasted_iota(jnp.float32, (EB, N), 1)
    a = af_ref[:]
    b = bf_ref[:]
    w = (jnp.where(jn < a, g0_ref[:], 0.0)
         + jnp.where((jn > a) & (jn < b), g1_ref[:], 0.0)
         + jnp.where(jn > b, g2_ref[:], 0.0))
    G = jnp.sum(H3r * w[:, :, None], axis=1)
    sw = jnp.sum(w, axis=1, keepdims=True)
    out_ref[:] = (jnp.dot(G, T24_ref[:], preferred_element_type=jnp.float32)
                  + b24_ref[:] * sw + P_s[pl.ds(i * EB, EB), :])


def kernel(x, edge_index, edge_features, num_nodes_batch, num_nodes,
           Wl, Wr, bl,
           m1W1, m1b1, m1W2, m1b2, m1W3, m1b3, m1W4, m1b4,
           m2W1, m2b1, m2W2, m2b2, m2W3, m2b3, m2W4, m2b4,
           Wlin, blin):
    N, D = x.shape
    E = edge_index.shape[1]
    EB = 64

    srcf = edge_index[0].astype(jnp.float32).reshape(E, 1)
    dstf = edge_index[1].astype(jnp.float32).reshape(E, 1)
    af = jnp.minimum(srcf, dstf)
    bf = jnp.maximum(srcf, dstf)
    ef = edge_features

    g = Wlin[0, 1:]
    g0 = jnp.pad(g, (0, 2)).reshape(1, N)
    g1 = jnp.pad(g, (1, 1)).reshape(1, N)
    g2 = jnp.pad(g, (2, 0)).reshape(1, N)

    def full(arr):
        nd = arr.ndim
        return pl.BlockSpec(arr.shape, lambda *_: (0,) * nd)

    ins = [
        x, srcf, dstf, ef, af, bf,
        Wl.T, Wr.T, bl.reshape(1, D),
        m1W1[:, :D].T, m1W1[:, D:2 * D].T, m1W1[:, 2 * D:].T, m1b1.reshape(1, -1),
        m1W2.T, m1b2.reshape(1, -1), m1W3.T, m1b3.reshape(1, -1),
        m1W4.T, m1b4.reshape(1, 1),
        m2W1[:, :D].T, m2W1[:, D:2 * D].T, m2W1[:, 2 * D:3 * D].T, m2W1[:, 3 * D:].T,
        m2b1.reshape(1, -1),
        m2W2.T, m2b2.reshape(1, -1), m2W3.T, m2b3.reshape(1, -1),
        m2W4.T, m2b4.reshape(1, 1),
        g0, g1, g2, Wlin[0:1, 0:1], blin.reshape(1, 1),
    ]
    specs = [full(a) for a in ins]
    specs[4] = pl.BlockSpec((EB, 1), lambda i: (i, 0))
    specs[5] = pl.BlockSpec((EB, 1), lambda i: (i, 0))

    pred = pl.pallas_call(
        functools.partial(_body, E=E, N=N, EB=EB),
        grid=(E // EB,),
        in_specs=specs,
        out_specs=pl.BlockSpec((EB, 1), lambda i: (i, 0)),
        out_shape=jax.ShapeDtypeStruct((E, 1), jnp.float32),
        scratch_shapes=[
            pltpu.VMEM((E, 256), jnp.float32),
            pltpu.VMEM((N, 256), jnp.float32),
            pltpu.VMEM((E, 1), jnp.float32),
        ],
        compiler_params=pltpu.CompilerParams(
            dimension_semantics=("arbitrary",),
        ),
    )(*ins)
    return pred


# speedup vs baseline: 11.6781x; 11.6781x over previous
import functools

import jax
import jax.numpy as jnp
from jax.experimental import pallas as pl
from jax.experimental.pallas import tpu as pltpu


def _dot(a, b):
    return jnp.dot(a, b, preferred_element_type=jnp.float32,
                   precision=jax.lax.Precision.HIGHEST)


def _dotd(a, b):
    return jnp.dot(a, b, preferred_element_type=jnp.float32)


def _relu(v):
    return jnp.maximum(v, 0.0)


def _body(x_ref, srcf_ref, dstf_ref, ef_ref, af_ref, bf_ref,
          WlT_ref, WrT_ref, bl_ref,
          W1a_ref, W1b_ref, W1c_ref, b11_ref, T12_ref, b12_ref,
          T13_ref, b13_ref, T14_ref, b14_ref,
          W2v_ref, W2u_ref, W2z_ref, W2e_ref, b21_ref, T22_ref, b22_ref,
          T23_ref, b23_ref, T24_ref, b24_ref,
          g0_ref, g1_ref, g2_ref, w00_ref, blin_ref,
          out_ref, A_s, Bz_s, P_s, *, E, N, EB):
    i = pl.program_id(0)

    @pl.when(i == 0)
    def _prologue():
        jN = jax.lax.broadcasted_iota(jnp.int32, (E, N), 1)
        oh_src = jnp.where(jN == srcf_ref[:], 1.0, 0.0)
        oh_dst = jnp.where(jN == dstf_ref[:], 1.0, 0.0)
        dn = (((0,), (0,)), ((), ()))
        M = jax.lax.dot_general(oh_dst, oh_src, dn,
                                preferred_element_type=jnp.float32,
                                precision=jax.lax.Precision.HIGHEST)
        ones_e = jnp.ones((E, 1), dtype=jnp.float32)
        cnt = jax.lax.dot_general(oh_dst, ones_e, dn,
                                  preferred_element_type=jnp.float32,
                                  precision=jax.lax.Precision.HIGHEST)
        recip = 1.0 / jnp.maximum(cnt, 1.0)
        h = x_ref[:]
        for _ in range(3):
            mean = _dot(M, h) * recip
            h = _relu(_dotd(mean, WlT_ref[:])
                      + bl_ref[:]
                      + _dotd(h, WrT_ref[:]))
        ve = _dot(oh_src, h)
        ue = _dot(oh_dst, h)
        efv = ef_ref[:]
        q = _relu(_dotd(ve, W1a_ref[:])
                  + _dotd(ue, W1b_ref[:])
                  + _dotd(efv, W1c_ref[:])
                  + b11_ref[:])
        q = _relu(_dotd(q, T12_ref[:]) + b12_ref[:])
        q = _relu(_dotd(q, T13_ref[:]) + b13_ref[:])
        pno = _dotd(q, T14_ref[:]) + b14_ref[:]
        P_s[:] = pno * w00_ref[:] + blin_ref[:]
        A_s[:] = (_dotd(ve, W2v_ref[:])
                  + _dotd(ue, W2u_ref[:])
                  + _dotd(efv, W2e_ref[:])
                  + b21_ref[:])
        Bz_s[:] = _dotd(h, W2z_ref[:])

    Ab = A_s[pl.ds(i * EB, EB), :]
    X = _relu(Ab[:, None, :] + Bz_s[:][None, :, :])
    X2 = X.reshape(EB * N, 256)
    H2 = _relu(_dotd(X2, T22_ref[:]) + b22_ref[:])
    H3 = _relu(_dotd(H2, T23_ref[:]) + b23_ref[:])
    H3r = H3.reshape(EB, N, 64)
    jn = jax.lax.broadcasted_iota(jnp.int32, (EB, N), 1)
    a = af_ref[:]
    b = bf_ref[:]
    w = (jnp.where(jn < a, g0_ref[:], 0.0)
         + jnp.where((jn > a) & (jn < b), g1_ref[:], 0.0)
         + jnp.where(jn > b, g2_ref[:], 0.0))
    G = jnp.sum(H3r * w[:, :, None], axis=1)
    sw = jnp.sum(w, axis=1, keepdims=True)
    out_ref[:] = (_dotd(G, T24_ref[:])
                  + b24_ref[:] * sw + P_s[pl.ds(i * EB, EB), :])


def kernel(x, edge_index, edge_features, num_nodes_batch, num_nodes,
           Wl, Wr, bl,
           m1W1, m1b1, m1W2, m1b2, m1W3, m1b3, m1W4, m1b4,
           m2W1, m2b1, m2W2, m2b2, m2W3, m2b3, m2W4, m2b4,
           Wlin, blin):
    N, D = x.shape
    E = edge_index.shape[1]
    EB = 64

    srcf = edge_index[0].astype(jnp.int32).reshape(E, 1)
    dstf = edge_index[1].astype(jnp.int32).reshape(E, 1)
    af = jnp.minimum(srcf, dstf)
    bf = jnp.maximum(srcf, dstf)
    ef = edge_features

    g = Wlin[0, 1:]
    g0 = jnp.pad(g, (0, 2)).reshape(1, N)
    g1 = jnp.pad(g, (1, 1)).reshape(1, N)
    g2 = jnp.pad(g, (2, 0)).reshape(1, N)

    def full(arr):
        nd = arr.ndim
        return pl.BlockSpec(arr.shape, lambda *_: (0,) * nd)

    ins = [
        x, srcf, dstf, ef, af, bf,
        Wl.T, Wr.T, bl.reshape(1, D),
        m1W1[:, :D].T, m1W1[:, D:2 * D].T, m1W1[:, 2 * D:].T, m1b1.reshape(1, -1),
        m1W2.T, m1b2.reshape(1, -1), m1W3.T, m1b3.reshape(1, -1),
        m1W4.T, m1b4.reshape(1, 1),
        m2W1[:, :D].T, m2W1[:, D:2 * D].T, m2W1[:, 2 * D:3 * D].T, m2W1[:, 3 * D:].T,
        m2b1.reshape(1, -1),
        m2W2.T, m2b2.reshape(1, -1), m2W3.T, m2b3.reshape(1, -1),
        m2W4.T, m2b4.reshape(1, 1),
        g0, g1, g2, Wlin[0:1, 0:1], blin.reshape(1, 1),
    ]
    specs = [full(a) for a in ins]
    specs[4] = pl.BlockSpec((EB, 1), lambda i: (i, 0))
    specs[5] = pl.BlockSpec((EB, 1), lambda i: (i, 0))

    pred = pl.pallas_call(
        functools.partial(_body, E=E, N=N, EB=EB),
        grid=(E // EB,),
        in_specs=specs,
        out_specs=pl.BlockSpec((EB, 1), lambda i: (i, 0)),
        out_shape=jax.ShapeDtypeStruct((E, 1), jnp.float32),
        scratch_shapes=[
            pltpu.VMEM((E, 256), jnp.float32),
            pltpu.VMEM((N, 256), jnp.float32),
            pltpu.VMEM((E, 1), jnp.float32),
        ],
        compiler_params=pltpu.CompilerParams(
            dimension_semantics=("arbitrary",),
        ),
    )(*ins)
    return pred


# speedup vs baseline: 12.0457x; 1.0315x over previous
import functools

import jax
import jax.numpy as jnp
from jax.experimental import pallas as pl
from jax.experimental.pallas import tpu as pltpu


def _dot(a, b):
    return jnp.dot(a, b, preferred_element_type=jnp.float32,
                   precision=jax.lax.Precision.HIGHEST)


def _dotd(a, b):
    return jnp.dot(a, b, preferred_element_type=jnp.float32)


def _relu(v):
    return jnp.maximum(v, 0.0)


def _body(x_ref, srcf_ref, dstf_ref, ef_ref, af_ref, bf_ref,
          WlT_ref, WrT_ref, bl_ref,
          W1a_ref, W1b_ref, W1c_ref, b11_ref, T12_ref, b12_ref,
          T13_ref, b13_ref, T14_ref, b14_ref,
          W2v_ref, W2u_ref, W2z_ref, W2e_ref, b21_ref, T22_ref, b22_ref,
          T23_ref, b23_ref, T24_ref, b24_ref,
          g0_ref, g1_ref, g2_ref, w00_ref, blin_ref,
          out_ref, A_s, Bz_s, P_s, *, E, N, EB):
    i = pl.program_id(0)

    @pl.when(i == 0)
    def _prologue():
        jN = jax.lax.broadcasted_iota(jnp.int32, (E, N), 1)
        oh_src = jnp.where(jN == srcf_ref[:], 1.0, 0.0)
        oh_dst = jnp.where(jN == dstf_ref[:], 1.0, 0.0)
        dn = (((0,), (0,)), ((), ()))
        M = jax.lax.dot_general(oh_dst, oh_src, dn,
                                preferred_element_type=jnp.float32,
                                precision=jax.lax.Precision.HIGHEST)
        ones_e = jnp.ones((E, 1), dtype=jnp.float32)
        cnt = jax.lax.dot_general(oh_dst, ones_e, dn,
                                  preferred_element_type=jnp.float32,
                                  precision=jax.lax.Precision.HIGHEST)
        recip = 1.0 / jnp.maximum(cnt, 1.0)
        h = x_ref[:]
        for _ in range(3):
            mean = _dot(M, h) * recip
            h = _relu(_dotd(mean, WlT_ref[:])
                      + bl_ref[:]
                      + _dotd(h, WrT_ref[:]))
        ve = _dot(oh_src, h)
        ue = _dot(oh_dst, h)
        efv = ef_ref[:]
        q = _relu(_dotd(ve, W1a_ref[:])
                  + _dotd(ue, W1b_ref[:])
                  + _dotd(efv, W1c_ref[:])
                  + b11_ref[:])
        q = _relu(_dotd(q, T12_ref[:]) + b12_ref[:])
        q = _relu(_dotd(q, T13_ref[:]) + b13_ref[:])
        pno = _dotd(q, T14_ref[:]) + b14_ref[:]
        P_s[:] = pno * w00_ref[:] + blin_ref[:]
        A_s[:] = (_dotd(ve, W2v_ref[:])
                  + _dotd(ue, W2u_ref[:])
                  + _dotd(efv, W2e_ref[:])
                  + b21_ref[:])
        Bz_s[:] = _dotd(h, W2z_ref[:])

    Ab = A_s[pl.ds(i * EB, EB), :]
    X = _relu(Ab[:, None, :] + Bz_s[:][None, :, :])
    X2 = X.reshape(EB * N, 256)
    H2 = _relu(_dotd(X2, T22_ref[:]) + b22_ref[:])
    H3 = _relu(_dotd(H2, T23_ref[:]) + b23_ref[:])
    H3r = H3.reshape(EB, N, 64)
    jn = jax.lax.broadcasted_iota(jnp.int32, (EB, N), 1)
    a = af_ref[:]
    b = bf_ref[:]
    w = (jnp.where(jn < a, g0_ref[:], 0.0)
         + jnp.where((jn > a) & (jn < b), g1_ref[:], 0.0)
         + jnp.where(jn > b, g2_ref[:], 0.0))
    G = jnp.sum(H3r * w[:, :, None], axis=1)
    sw = jnp.sum(w, axis=1, keepdims=True)
    out_ref[:] = (_dotd(G, T24_ref[:])
                  + b24_ref[:] * sw + P_s[pl.ds(i * EB, EB), :])


def kernel(x, edge_index, edge_features, num_nodes_batch, num_nodes,
           Wl, Wr, bl,
           m1W1, m1b1, m1W2, m1b2, m1W3, m1b3, m1W4, m1b4,
           m2W1, m2b1, m2W2, m2b2, m2W3, m2b3, m2W4, m2b4,
           Wlin, blin):
    N, D = x.shape
    E = edge_index.shape[1]
    EB = 128

    srcf = edge_index[0].astype(jnp.int32).reshape(E, 1)
    dstf = edge_index[1].astype(jnp.int32).reshape(E, 1)
    af = jnp.minimum(srcf, dstf)
    bf = jnp.maximum(srcf, dstf)
    ef = edge_features

    g = Wlin[0, 1:]
    g0 = jnp.pad(g, (0, 2)).reshape(1, N)
    g1 = jnp.pad(g, (1, 1)).reshape(1, N)
    g2 = jnp.pad(g, (2, 0)).reshape(1, N)

    def full(arr):
        nd = arr.ndim
        return pl.BlockSpec(arr.shape, lambda *_: (0,) * nd)

    ins = [
        x, srcf, dstf, ef, af, bf,
        Wl.T, Wr.T, bl.reshape(1, D),
        m1W1[:, :D].T, m1W1[:, D:2 * D].T, m1W1[:, 2 * D:].T, m1b1.reshape(1, -1),
        m1W2.T, m1b2.reshape(1, -1), m1W3.T, m1b3.reshape(1, -1),
        m1W4.T, m1b4.reshape(1, 1),
        m2W1[:, :D].T, m2W1[:, D:2 * D].T, m2W1[:, 2 * D:3 * D].T, m2W1[:, 3 * D:].T,
        m2b1.reshape(1, -1),
        m2W2.T, m2b2.reshape(1, -1), m2W3.T, m2b3.reshape(1, -1),
        m2W4.T, m2b4.reshape(1, 1),
        g0, g1, g2, Wlin[0:1, 0:1], blin.reshape(1, 1),
    ]
    specs = [full(a) for a in ins]
    specs[4] = pl.BlockSpec((EB, 1), lambda i: (i, 0))
    specs[5] = pl.BlockSpec((EB, 1), lambda i: (i, 0))

    pred = pl.pallas_call(
        functools.partial(_body, E=E, N=N, EB=EB),
        grid=(E // EB,),
        in_specs=specs,
        out_specs=pl.BlockSpec((EB, 1), lambda i: (i, 0)),
        out_shape=jax.ShapeDtypeStruct((E, 1), jnp.float32),
        scratch_shapes=[
            pltpu.VMEM((E, 256), jnp.float32),
            pltpu.VMEM((N, 256), jnp.float32),
            pltpu.VMEM((E, 1), jnp.float32),
        ],
        compiler_params=pltpu.CompilerParams(
            dimension_semantics=("arbitrary",),
        ),
    )(*ins)
    return pred
